# baseline (speedup 1.0000x reference)
import functools

import jax
import jax.numpy as jnp
from jax import lax
from jax.experimental import pallas as pl
from jax.experimental.pallas import tpu as pltpu
from jax.experimental.pallas import tpu_sc as plsc

_NC = 2
_NS = 16
_NW = _NC * _NS
_L = 16


def _tc_tables(n: int, x_ref, w_ref, b_ref, s_ref, xo_ref):
    x = x_ref[...]
    row = lax.broadcasted_iota(jnp.int32, (2, 1), 0)
    badd = jnp.where(row == 0, b_ref[0, 0], jnp.float32(0))
    s_ref[:, pl.ds(0, n)] = (
        lax.dot_general(
            w_ref[...],
            x,
            dimension_numbers=(((1,), (1,)), ((), ())),
            preferred_element_type=jnp.float32,
        )
        + badd
    )
    xo_ref[...] = x


_BLK = 128


def _make_sc_score(n_pad: int, n_edges: int):
    nblk = n_edges // _BLK
    lo_blk, rem = divmod(nblk, _NW)
    main_cnt = lo_blk * _BLK
    tail_base0 = _NW * main_cnt

    def _sc_score(s_hbm, edge_hbm, out_hbm, eo_hbm, ts, exy, ov, sem_i, sem_o):
        wid = lax.axis_index("s") * _NC + lax.axis_index("c")
        base = wid * main_cnt
        cp_in = pltpu.async_copy(
            edge_hbm.at[:, pl.ds(base, main_cnt)],
            exy.at[:, pl.ds(0, main_cnt)],
            sem_i,
        )
        pltpu.sync_copy(s_hbm, ts)
        cp_in.wait()
        cp_out = pltpu.async_copy(
            exy.at[:, pl.ds(0, main_cnt)],
            eo_hbm.at[:, pl.ds(base, main_cnt)],
            sem_o,
        )
        row0 = jnp.zeros((_L,), jnp.int32)
        row1 = jnp.ones((_L,), jnp.int32)

        def score(cnt, unroll):
            @plsc.parallel_loop(0, cnt, step=_L, unroll=unroll)
            def _body(off):
                ivs = exy[0, pl.ds(off, _L)]
                ivd = exy[1, pl.ds(off, _L)]
                g1 = plsc.load_gather(ts, [row0, ivs])
                g2 = plsc.load_gather(ts, [row1, ivd])
                z = g1 + g2
                ov[pl.ds(off, _L)] = 1.0 / (1.0 + jnp.exp(-z)) + 0.3

        score(main_cnt, 32)
        pltpu.sync_copy(ov.at[pl.ds(0, main_cnt)], out_hbm.at[pl.ds(base, main_cnt)])
        cp_out.wait()
        if rem:
            def tail():
                tb = tail_base0 + wid * _BLK
                pltpu.sync_copy(
                    edge_hbm.at[:, pl.ds(tb, _BLK)], exy.at[:, pl.ds(0, _BLK)]
                )
                score(_BLK, 8)
                pltpu.sync_copy(ov.at[pl.ds(0, _BLK)], out_hbm.at[pl.ds(tb, _BLK)])
                pltpu.sync_copy(
                    exy.at[:, pl.ds(0, _BLK)], eo_hbm.at[:, pl.ds(tb, _BLK)]
                )

            pl.when(wid < rem)(tail)

    sc_call = functools.partial(
        pl.kernel,
        out_type=[
            jax.ShapeDtypeStruct((n_edges,), jnp.float32),
            jax.ShapeDtypeStruct((2, n_edges), jnp.int32),
        ],
        mesh=plsc.VectorSubcoreMesh(core_axis_name="c", subcore_axis_name="s"),
        compiler_params=pltpu.CompilerParams(needs_layout_passes=False),
        scratch_types=[
            pltpu.VMEM((2, n_pad), jnp.float32),
            pltpu.VMEM((2, main_cnt), jnp.int32),
            pltpu.VMEM((main_cnt,), jnp.float32),
            pltpu.SemaphoreType.DMA,
            pltpu.SemaphoreType.DMA,
        ],
    )(_sc_score)
    return sc_call


def kernel(x, edge_index, W, b):
    n, c = x.shape
    n_edges = edge_index.shape[1]
    n_pad = -(-n // _BLK) * _BLK

    s, x_out = pl.pallas_call(
        functools.partial(_tc_tables, n),
        out_shape=[
            jax.ShapeDtypeStruct((2, n_pad), jnp.float32),
            jax.ShapeDtypeStruct(x.shape, x.dtype),
        ],
    )(x, W.reshape(2, c), b.reshape(1, 1))

    ei = edge_index if edge_index.dtype == jnp.int32 else (
        edge_index.astype(jnp.int32))
    e, e_out = _make_sc_score(n_pad, n_edges)(s, ei)
    return (x_out, e_out, e)


# speedup vs baseline: 1.1062x; 1.1062x over previous
import functools

import jax
import jax.numpy as jnp
from jax import lax
from jax.experimental import pallas as pl
from jax.experimental.pallas import tpu as pltpu
from jax.experimental.pallas import tpu_sc as plsc

_NC = 2
_NS = 16
_NW = _NC * _NS
_L = 16


def _tc_tables(n: int, x_ref, w_ref, b_ref, s_ref, xo_ref):
    x = x_ref[...]
    row = lax.broadcasted_iota(jnp.int32, (2, 1), 0)
    badd = jnp.where(row == 0, b_ref[0, 0], jnp.float32(0))
    s_ref[:, pl.ds(0, n)] = (
        lax.dot_general(
            w_ref[...],
            x,
            dimension_numbers=(((1,), (1,)), ((), ())),
            preferred_element_type=jnp.float32,
        )
        + badd
    )
    xo_ref[...] = x


_BLK = 128


def _make_sc_score(n_pad: int, n_edges: int):
    nblk = n_edges // _BLK
    lo_blk, rem = divmod(nblk, _NW)
    main_cnt = lo_blk * _BLK
    tail_base0 = _NW * main_cnt

    def _sc_score(s_hbm, edge_hbm, out_hbm, eo_hbm, ts, exy, ov, sem_i, sem_o):
        wid = lax.axis_index("s") * _NC + lax.axis_index("c")
        base = wid * main_cnt
        cp_in = pltpu.async_copy(
            edge_hbm.at[:, pl.ds(base, main_cnt)],
            exy.at[:, pl.ds(0, main_cnt)],
            sem_i,
        )
        pltpu.sync_copy(s_hbm, ts)
        cp_in.wait()
        cp_out = pltpu.async_copy(
            exy.at[:, pl.ds(0, main_cnt)],
            eo_hbm.at[:, pl.ds(base, main_cnt)],
            sem_o,
        )
        row0 = jnp.zeros((_L,), jnp.int32)
        row1 = jnp.ones((_L,), jnp.int32)

        def score(cnt, unroll):
            @plsc.parallel_loop(0, cnt, step=_L, unroll=unroll)
            def _body(off):
                ivs = exy[0, pl.ds(off, _L)]
                ivd = exy[1, pl.ds(off, _L)]
                g1 = plsc.load_gather(ts, [row0, ivs])
                g2 = plsc.load_gather(ts, [row1, ivd])
                z = g1 + g2
                ov[pl.ds(off, _L)] = 1.0 / (1.0 + jnp.exp(-z)) + 0.3

        score(main_cnt, 16)
        pltpu.sync_copy(ov.at[pl.ds(0, main_cnt)], out_hbm.at[pl.ds(base, main_cnt)])
        cp_out.wait()
        if rem:
            def tail():
                tb = tail_base0 + wid * _BLK
                pltpu.sync_copy(
                    edge_hbm.at[:, pl.ds(tb, _BLK)], exy.at[:, pl.ds(0, _BLK)]
                )
                score(_BLK, 8)
                pltpu.sync_copy(ov.at[pl.ds(0, _BLK)], out_hbm.at[pl.ds(tb, _BLK)])
                pltpu.sync_copy(
                    exy.at[:, pl.ds(0, _BLK)], eo_hbm.at[:, pl.ds(tb, _BLK)]
                )

            pl.when(wid < rem)(tail)

    sc_call = functools.partial(
        pl.kernel,
        out_type=[
            jax.ShapeDtypeStruct((n_edges,), jnp.float32),
            jax.ShapeDtypeStruct((2, n_edges), jnp.int32),
        ],
        mesh=plsc.VectorSubcoreMesh(core_axis_name="c", subcore_axis_name="s"),
        compiler_params=pltpu.CompilerParams(needs_layout_passes=False),
        scratch_types=[
            pltpu.VMEM((2, n_pad), jnp.float32),
            pltpu.VMEM((2, main_cnt), jnp.int32),
            pltpu.VMEM((main_cnt,), jnp.float32),
            pltpu.SemaphoreType.DMA,
            pltpu.SemaphoreType.DMA,
        ],
    )(_sc_score)
    return sc_call


def kernel(x, edge_index, W, b):
    n, c = x.shape
    n_edges = edge_index.shape[1]
    n_pad = -(-n // _BLK) * _BLK

    s, x_out = pl.pallas_call(
        functools.partial(_tc_tables, n),
        out_shape=[
            jax.ShapeDtypeStruct((2, n_pad), jnp.float32),
            jax.ShapeDtypeStruct(x.shape, x.dtype),
        ],
    )(x, W.reshape(2, c), b.reshape(1, 1))

    ei = edge_index if edge_index.dtype == jnp.int32 else (
        edge_index.astype(jnp.int32))
    e, e_out = _make_sc_score(n_pad, n_edges)(s, ei)
    return (x_out, e_out, e)
